# baseline (speedup 1.0000x reference)
import functools

import jax
import jax.numpy as jnp
from jax import lax
from jax.experimental import pallas as pl
from jax.experimental.pallas import tpu as pltpu
from jax.experimental.pallas import tpu_sc as plsc

N = 10000
E = 320000
IN_CH = 128
HID = 64
OUT_CH = 64
D_EDGE = 4

NPAD = 10240
DAUG = 80
NTILES = 32
EPT = E // NTILES
C = 80
NCHUNK = EPT // C
TROWS = NPAD // 16

_EPS = 1e-16


def _feat_from_x_body(x_ref, w_ref, asv_ref, adv_ref, aug_ref, as_ref, ad_ref):
    xp = jnp.dot(x_ref[...], w_ref[...], preferred_element_type=jnp.float32)
    aug_ref[:, 0:HID] = xp
    aug_ref[:, HID:HID + 1] = jnp.ones((xp.shape[0], 1), jnp.float32)
    aug_ref[:, HID + 1:DAUG] = jnp.zeros((xp.shape[0], DAUG - HID - 1), jnp.float32)
    as_ref[...] = jnp.dot(xp, asv_ref[...], preferred_element_type=jnp.float32)[:, 0]
    ad_ref[...] = jnp.dot(xp, adv_ref[...], preferred_element_type=jnp.float32)[:, 0]


def _feat1(x, w, asv, adv, blk=2048):
    grid = NPAD // blk
    return pl.pallas_call(
        _feat_from_x_body,
        grid=(grid,),
        in_specs=[
            pl.BlockSpec((blk, IN_CH), lambda i: (i, 0)),
            pl.BlockSpec((IN_CH, HID), lambda i: (0, 0)),
            pl.BlockSpec((HID, 1), lambda i: (0, 0)),
            pl.BlockSpec((HID, 1), lambda i: (0, 0)),
        ],
        out_specs=[
            pl.BlockSpec((blk, DAUG), lambda i: (i, 0)),
            pl.BlockSpec((blk,), lambda i: (i,)),
            pl.BlockSpec((blk,), lambda i: (i,)),
        ],
        out_shape=[
            jax.ShapeDtypeStruct((NPAD, DAUG), jnp.float32),
            jax.ShapeDtypeStruct((NPAD,), jnp.float32),
            jax.ShapeDtypeStruct((NPAD,), jnp.float32),
        ],
    )(x, w, asv, adv)


def _feat_from_acc_body(acc_ref, b_ref, w_ref, asv_ref, adv_ref,
                        aug_ref, as_ref, ad_ref):
    num = acc_ref[0, :, 0:HID] + acc_ref[1, :, 0:HID]
    den = acc_ref[0, :, HID:HID + 1] + acc_ref[1, :, HID:HID + 1]
    h = num / (den + _EPS) + b_ref[...]
    h = jnp.where(h > 0, h, jnp.exp(h) - 1.0)
    xp = jnp.dot(h, w_ref[...], preferred_element_type=jnp.float32)
    aug_ref[:, 0:HID] = xp
    aug_ref[:, HID:HID + 1] = jnp.ones((xp.shape[0], 1), jnp.float32)
    aug_ref[:, HID + 1:DAUG] = jnp.zeros((xp.shape[0], DAUG - HID - 1), jnp.float32)
    as_ref[...] = jnp.dot(xp, asv_ref[...], preferred_element_type=jnp.float32)[:, 0]
    ad_ref[...] = jnp.dot(xp, adv_ref[...], preferred_element_type=jnp.float32)[:, 0]


def _feat2(acc, b, w, asv, adv, blk=2048):
    grid = NPAD // blk
    return pl.pallas_call(
        _feat_from_acc_body,
        grid=(grid,),
        in_specs=[
            pl.BlockSpec((2, blk, DAUG), lambda i: (0, i, 0)),
            pl.BlockSpec((1, HID), lambda i: (0, 0)),
            pl.BlockSpec((HID, HID), lambda i: (0, 0)),
            pl.BlockSpec((HID, 1), lambda i: (0, 0)),
            pl.BlockSpec((HID, 1), lambda i: (0, 0)),
        ],
        out_specs=[
            pl.BlockSpec((blk, DAUG), lambda i: (i, 0)),
            pl.BlockSpec((blk,), lambda i: (i,)),
            pl.BlockSpec((blk,), lambda i: (i,)),
        ],
        out_shape=[
            jax.ShapeDtypeStruct((NPAD, DAUG), jnp.float32),
            jax.ShapeDtypeStruct((NPAD,), jnp.float32),
            jax.ShapeDtypeStruct((NPAD,), jnp.float32),
        ],
    )(acc, b, w, asv, adv)


def _final_body(acc_ref, b_ref, wl_ref, bl_ref, out_ref):
    num = acc_ref[0, :, 0:HID] + acc_ref[1, :, 0:HID]
    den = acc_ref[0, :, HID:HID + 1] + acc_ref[1, :, HID:HID + 1]
    h = num / (den + _EPS) + b_ref[...]
    logit = jnp.dot(h, wl_ref[...], preferred_element_type=jnp.float32) + bl_ref[...]
    out_ref[...] = jax.nn.sigmoid(logit)


def _final(acc, b, wl, bl, blk=2048):
    grid = NPAD // blk
    return pl.pallas_call(
        _final_body,
        grid=(grid,),
        in_specs=[
            pl.BlockSpec((2, blk, DAUG), lambda i: (0, i, 0)),
            pl.BlockSpec((1, OUT_CH), lambda i: (0, 0)),
            pl.BlockSpec((OUT_CH, 1), lambda i: (0, 0)),
            pl.BlockSpec((1, 1), lambda i: (0, 0)),
        ],
        out_specs=pl.BlockSpec((blk, 1), lambda i: (i, 0)),
        out_shape=jax.ShapeDtypeStruct((NPAD, 1), jnp.float32),
    )(acc, b, wl, bl)


def _sc_edge_body(xp_hbm, asrc_hbm, adst_hbm, ei_hbm, ea_hbm,
                  we_hbm, aev_hbm, out_hbm, asrc_t, adst_t, sidx_t, didx_t,
                  we_t, aev_t, wbuf, rows0, rows1, rows2, eab0, eab1, eab2,
                  dc0, dc1, dc2, acc_sp, bsem, gs0, gs1, gs2, ss0, ss1, ss2):
    core = lax.axis_index("c")
    sub = lax.axis_index("s")
    wid = core * 16 + sub
    ebase = wid * EPT

    rows = [rows0, rows1, rows2]
    eabs = [eab0, eab1, eab2]
    dcs = [dc0, dc1, dc2]
    gsems = [gs0, gs1, gs2]
    ssems = [ss0, ss1, ss2]

    zeros16 = jnp.zeros((16,), jnp.float32)
    iota16 = lax.iota(jnp.int32, 16)

    bulk = [
        pltpu.async_copy(ei_hbm.at[0, pl.ds(ebase, EPT)], sidx_t, bsem),
        pltpu.async_copy(ei_hbm.at[1, pl.ds(ebase, EPT)], didx_t, bsem),
        pltpu.async_copy(asrc_hbm, asrc_t, bsem),
        pltpu.async_copy(adst_hbm, adst_t, bsem),
        pltpu.async_copy(we_hbm, we_t, bsem),
        pltpu.async_copy(aev_hbm, aev_t, bsem),
    ]

    def _zrow(i, _):
        for v in range(DAUG // 16):
            rows0[i, pl.ds(v * 16, 16)] = zeros16
        return 0
    lax.fori_loop(0, C, _zrow, 0)

    def _zinit(k, _):
        pltpu.sync_copy(rows0, acc_sp.at[pl.ds(sub * TROWS + k * C, C), :])
        return 0
    lax.fori_loop(0, TROWS // C, _zinit, 0)

    for d in bulk:
        d.wait()

    wv = []
    for k in range(D_EDGE):
        acc16 = we_t[pl.ds(k * HID, 16)] * aev_t[pl.ds(0, 16)]
        for q in range(1, HID // 16):
            acc16 = acc16 + (we_t[pl.ds(k * HID + q * 16, 16)]
                             * aev_t[pl.ds(q * 16, 16)])
        wv.append(jnp.zeros((16,), jnp.float32) + jnp.sum(acc16))

    plsc.subcore_barrier()

    def _gather_descs(g, b):
        return [
            pltpu.make_async_copy(
                xp_hbm.at[sidx_t.at[pl.ds(g * C, C)]], rows[b], gsems[b]),
            pltpu.make_async_copy(
                ea_hbm.at[pl.ds(ebase + g * C, C), :], eabs[b], gsems[b]),
        ]

    def _scatter_desc(b):
        return pltpu.make_async_copy(rows[b], acc_sp.at[dcs[b]], ssems[b])

    def _step(g, b, wait_prev_scatter, issue_next_gather):
        if wait_prev_scatter:
            _scatter_desc((b + 2) % 3).wait()
        if issue_next_gather:
            for d in _gather_descs(g + 2, (b + 2) % 3):
                d.start()
        for d in _gather_descs(g, b):
            d.wait()
        dc = dcs[b]
        eb = eabs[b]
        for j in range(C // 16):
            d16 = didx_t[pl.ds(g * C + j * 16, 16)]
            dc[pl.ds(j * 16, 16)] = d16
            s16 = sidx_t[pl.ds(g * C + j * 16, 16)]
            e16 = j * 16 + iota16
            zi16 = jnp.zeros((16,), jnp.int32)
            ae16 = plsc.load_gather(eb, [e16, zi16]) * wv[0]
            for k in range(1, D_EDGE):
                ae16 = ae16 + plsc.load_gather(eb, [e16, zi16 + k]) * wv[k]
            a = (plsc.load_gather(asrc_t, [s16])
                 + plsc.load_gather(adst_t, [d16])
                 + ae16)
            a = jnp.where(a > 0, a, 0.2 * a)
            wbuf[pl.ds(j * 16, 16)] = jnp.exp(a)
        rb = rows[b]

        def _scale(i2, _):
            for k in range(2):
                i = i2 * 2 + k
                wb = plsc.load_gather(wbuf, [jnp.zeros((16,), jnp.int32) + i])
                for v in range(DAUG // 16):
                    rb[i, pl.ds(v * 16, 16)] = rb[i, pl.ds(v * 16, 16)] * wb
            return 0
        lax.fori_loop(0, C // 2, _scale, 0)
        pltpu.async_copy(rows[b], acc_sp.at[dcs[b]], ssems[b], add=True)

    for d in _gather_descs(0, 0):
        d.start()
    for d in _gather_descs(1, 1):
        d.start()
    _step(0, 0, False, True)

    def _main(t, _):
        g = 3 * t + 1
        _step(g, 1, True, True)
        _step(g + 1, 2, True, True)
        _step(g + 2, 0, True, True)
        return 0
    lax.fori_loop(0, (NCHUNK - 5) // 3, _main, 0)

    _step(NCHUNK - 4, 1, True, True)
    _step(NCHUNK - 3, 2, True, True)
    _step(NCHUNK - 2, 0, True, False)
    _step(NCHUNK - 1, 1, True, False)
    _scatter_desc(1).wait()

    plsc.subcore_barrier()

    pltpu.sync_copy(acc_sp.at[pl.ds(sub * TROWS, TROWS), :],
                    out_hbm.at[core, pl.ds(sub * TROWS, TROWS), :])


def _sc_edge(xp_aug, asrc, adst, edge_index, edge_attr, we_flat, aev):
    mesh = plsc.VectorSubcoreMesh(core_axis_name="c", subcore_axis_name="s")
    f = pl.kernel(
        _sc_edge_body,
        out_type=jax.ShapeDtypeStruct((2, NPAD, DAUG), jnp.float32),
        mesh=mesh,
        compiler_params=pltpu.CompilerParams(needs_layout_passes=False,
                                             use_tc_tiling_on_sc=False),
        scratch_types=[
            pltpu.VMEM((NPAD,), jnp.float32),
            pltpu.VMEM((NPAD,), jnp.float32),
            pltpu.VMEM((EPT,), jnp.int32),
            pltpu.VMEM((EPT,), jnp.int32),
            pltpu.VMEM((D_EDGE * HID,), jnp.float32),
            pltpu.VMEM((HID,), jnp.float32),
            pltpu.VMEM((C,), jnp.float32),
            pltpu.VMEM((C, DAUG), jnp.float32),
            pltpu.VMEM((C, DAUG), jnp.float32),
            pltpu.VMEM((C, DAUG), jnp.float32),
            pltpu.VMEM((C, D_EDGE), jnp.float32),
            pltpu.VMEM((C, D_EDGE), jnp.float32),
            pltpu.VMEM((C, D_EDGE), jnp.float32),
            pltpu.VMEM((C,), jnp.int32),
            pltpu.VMEM((C,), jnp.int32),
            pltpu.VMEM((C,), jnp.int32),
            pltpu.VMEM_SHARED((NPAD, DAUG), jnp.float32),
            pltpu.SemaphoreType.DMA,
            pltpu.SemaphoreType.DMA,
            pltpu.SemaphoreType.DMA,
            pltpu.SemaphoreType.DMA,
            pltpu.SemaphoreType.DMA,
            pltpu.SemaphoreType.DMA,
            pltpu.SemaphoreType.DMA,
        ],
    )
    return f(xp_aug, asrc, adst, edge_index, edge_attr, we_flat, aev)


def kernel(x, edge_index, edge_attr, W1, as1, ad1, We1, ae1, b1,
           W2, as2, ad2, We2, ae2, b2, Wl, bl):
    xpad = jnp.pad(x, ((0, NPAD - N), (0, 0)))

    aug1, asr1, adr1 = _feat1(xpad, W1, as1.reshape(HID, 1), ad1.reshape(HID, 1))
    acc1 = _sc_edge(aug1, asr1, adr1, edge_index, edge_attr,
                    We1.reshape(D_EDGE * HID), ae1.reshape(HID))

    aug2, asr2, adr2 = _feat2(acc1, b1.reshape(1, HID), W2,
                              as2.reshape(OUT_CH, 1), ad2.reshape(OUT_CH, 1))
    acc2 = _sc_edge(aug2, asr2, adr2, edge_index, edge_attr,
                    We2.reshape(D_EDGE * OUT_CH), ae2.reshape(OUT_CH))

    out = _final(acc2, b2.reshape(1, OUT_CH), Wl, bl.reshape(1, 1))
    return out[:N]


# speedup vs baseline: 1.1157x; 1.1157x over previous
import functools

import jax
import jax.numpy as jnp
from jax import lax
from jax.experimental import pallas as pl
from jax.experimental.pallas import tpu as pltpu
from jax.experimental.pallas import tpu_sc as plsc

N = 10000
E = 320000
IN_CH = 128
HID = 64
OUT_CH = 64
D_EDGE = 4

NPAD = 10240
DAUG = 80
NTILES = 32
EPT = E // NTILES
C = 80
NCHUNK = EPT // C
TROWS = NPAD // 16

_EPS = 1e-16


def _feat_from_x_body(x_ref, w_ref, asv_ref, adv_ref, aug_ref, as_ref, ad_ref):
    xp = jnp.dot(x_ref[...], w_ref[...], preferred_element_type=jnp.float32)
    aug_ref[:, 0:HID] = xp
    aug_ref[:, HID:HID + 1] = jnp.ones((xp.shape[0], 1), jnp.float32)
    aug_ref[:, HID + 1:DAUG] = jnp.zeros((xp.shape[0], DAUG - HID - 1), jnp.float32)
    as_ref[...] = jnp.dot(xp, asv_ref[...], preferred_element_type=jnp.float32)[:, 0]
    ad_ref[...] = jnp.dot(xp, adv_ref[...], preferred_element_type=jnp.float32)[:, 0]


def _feat1(x, w, asv, adv, blk=2048):
    grid = NPAD // blk
    return pl.pallas_call(
        _feat_from_x_body,
        grid=(grid,),
        in_specs=[
            pl.BlockSpec((blk, IN_CH), lambda i: (i, 0)),
            pl.BlockSpec((IN_CH, HID), lambda i: (0, 0)),
            pl.BlockSpec((HID, 1), lambda i: (0, 0)),
            pl.BlockSpec((HID, 1), lambda i: (0, 0)),
        ],
        out_specs=[
            pl.BlockSpec((blk, DAUG), lambda i: (i, 0)),
            pl.BlockSpec((blk,), lambda i: (i,)),
            pl.BlockSpec((blk,), lambda i: (i,)),
        ],
        out_shape=[
            jax.ShapeDtypeStruct((NPAD, DAUG), jnp.float32),
            jax.ShapeDtypeStruct((NPAD,), jnp.float32),
            jax.ShapeDtypeStruct((NPAD,), jnp.float32),
        ],
    )(x, w, asv, adv)


def _feat_from_acc_body(acc_ref, b_ref, w_ref, asv_ref, adv_ref,
                        aug_ref, as_ref, ad_ref):
    num = acc_ref[0, :, 0:HID] + acc_ref[1, :, 0:HID]
    den = acc_ref[0, :, HID:HID + 1] + acc_ref[1, :, HID:HID + 1]
    h = num / (den + _EPS) + b_ref[...]
    h = jnp.where(h > 0, h, jnp.exp(h) - 1.0)
    xp = jnp.dot(h, w_ref[...], preferred_element_type=jnp.float32)
    aug_ref[:, 0:HID] = xp
    aug_ref[:, HID:HID + 1] = jnp.ones((xp.shape[0], 1), jnp.float32)
    aug_ref[:, HID + 1:DAUG] = jnp.zeros((xp.shape[0], DAUG - HID - 1), jnp.float32)
    as_ref[...] = jnp.dot(xp, asv_ref[...], preferred_element_type=jnp.float32)[:, 0]
    ad_ref[...] = jnp.dot(xp, adv_ref[...], preferred_element_type=jnp.float32)[:, 0]


def _feat2(acc, b, w, asv, adv, blk=2048):
    grid = NPAD // blk
    return pl.pallas_call(
        _feat_from_acc_body,
        grid=(grid,),
        in_specs=[
            pl.BlockSpec((2, blk, DAUG), lambda i: (0, i, 0)),
            pl.BlockSpec((1, HID), lambda i: (0, 0)),
            pl.BlockSpec((HID, HID), lambda i: (0, 0)),
            pl.BlockSpec((HID, 1), lambda i: (0, 0)),
            pl.BlockSpec((HID, 1), lambda i: (0, 0)),
        ],
        out_specs=[
            pl.BlockSpec((blk, DAUG), lambda i: (i, 0)),
            pl.BlockSpec((blk,), lambda i: (i,)),
            pl.BlockSpec((blk,), lambda i: (i,)),
        ],
        out_shape=[
            jax.ShapeDtypeStruct((NPAD, DAUG), jnp.float32),
            jax.ShapeDtypeStruct((NPAD,), jnp.float32),
            jax.ShapeDtypeStruct((NPAD,), jnp.float32),
        ],
    )(acc, b, w, asv, adv)


def _final_body(acc_ref, b_ref, wl_ref, bl_ref, out_ref):
    num = acc_ref[0, :, 0:HID] + acc_ref[1, :, 0:HID]
    den = acc_ref[0, :, HID:HID + 1] + acc_ref[1, :, HID:HID + 1]
    h = num / (den + _EPS) + b_ref[...]
    logit = jnp.dot(h, wl_ref[...], preferred_element_type=jnp.float32) + bl_ref[...]
    out_ref[...] = jax.nn.sigmoid(logit)


def _final(acc, b, wl, bl, blk=2048):
    grid = NPAD // blk
    return pl.pallas_call(
        _final_body,
        grid=(grid,),
        in_specs=[
            pl.BlockSpec((2, blk, DAUG), lambda i: (0, i, 0)),
            pl.BlockSpec((1, OUT_CH), lambda i: (0, 0)),
            pl.BlockSpec((OUT_CH, 1), lambda i: (0, 0)),
            pl.BlockSpec((1, 1), lambda i: (0, 0)),
        ],
        out_specs=pl.BlockSpec((blk, 1), lambda i: (i, 0)),
        out_shape=jax.ShapeDtypeStruct((NPAD, 1), jnp.float32),
    )(acc, b, wl, bl)


def _sc_edge_body(xp_hbm, asrc_hbm, adst_hbm, src_hbm, dst_hbm, ea_hbm,
                  we_hbm, aev_hbm, out_hbm, asrc_t, adst_t, sidx_t, didx_t,
                  we_t, aev_t, wbuf, rows0, rows1, rows2, eab0, eab1, eab2,
                  dc0, dc1, dc2, acc_sp, bsem, gs0, gs1, gs2, ss0, ss1, ss2):
    core = lax.axis_index("c")
    sub = lax.axis_index("s")
    wid = core * 16 + sub
    ebase = wid * EPT

    rows = [rows0, rows1, rows2]
    eabs = [eab0, eab1, eab2]
    dcs = [dc0, dc1, dc2]
    gsems = [gs0, gs1, gs2]
    ssems = [ss0, ss1, ss2]

    zeros16 = jnp.zeros((16,), jnp.float32)
    iota16 = lax.iota(jnp.int32, 16)

    bulk = [
        pltpu.async_copy(src_hbm.at[pl.ds(ebase, EPT)], sidx_t, bsem),
        pltpu.async_copy(dst_hbm.at[pl.ds(ebase, EPT)], didx_t, bsem),
        pltpu.async_copy(asrc_hbm, asrc_t, bsem),
        pltpu.async_copy(adst_hbm, adst_t, bsem),
        pltpu.async_copy(we_hbm, we_t, bsem),
        pltpu.async_copy(aev_hbm, aev_t, bsem),
    ]

    def _zrow(i, _):
        for v in range(DAUG // 16):
            rows0[i, pl.ds(v * 16, 16)] = zeros16
        return 0
    lax.fori_loop(0, C, _zrow, 0)

    def _zinit(k, _):
        pltpu.sync_copy(rows0, acc_sp.at[pl.ds(sub * TROWS + k * C, C), :])
        return 0
    lax.fori_loop(0, TROWS // C, _zinit, 0)

    for d in bulk:
        d.wait()

    wv = []
    for k in range(D_EDGE):
        acc16 = we_t[pl.ds(k * HID, 16)] * aev_t[pl.ds(0, 16)]
        for q in range(1, HID // 16):
            acc16 = acc16 + (we_t[pl.ds(k * HID + q * 16, 16)]
                             * aev_t[pl.ds(q * 16, 16)])
        wv.append(jnp.zeros((16,), jnp.float32) + jnp.sum(acc16))

    plsc.subcore_barrier()

    def _gather_descs(g, b):
        return [
            pltpu.make_async_copy(
                xp_hbm.at[sidx_t.at[pl.ds(g * C, C)]], rows[b], gsems[b]),
            pltpu.make_async_copy(
                ea_hbm.at[pl.ds((ebase + g * C) * D_EDGE, C * D_EDGE)],
                eabs[b], gsems[b]),
        ]

    def _scatter_desc(b):
        return pltpu.make_async_copy(rows[b], acc_sp.at[dcs[b]], ssems[b])

    def _step(g, b, wait_prev_scatter, issue_next_gather):
        if wait_prev_scatter:
            _scatter_desc((b + 2) % 3).wait()
        if issue_next_gather:
            for d in _gather_descs(g + 2, (b + 2) % 3):
                d.start()
        for d in _gather_descs(g, b):
            d.wait()
        dc = dcs[b]
        eb = eabs[b]
        for j in range(C // 16):
            d16 = didx_t[pl.ds(g * C + j * 16, 16)]
            dc[pl.ds(j * 16, 16)] = d16
            s16 = sidx_t[pl.ds(g * C + j * 16, 16)]
            base4 = (j * 16 + iota16) * D_EDGE
            ae16 = plsc.load_gather(eb, [base4]) * wv[0]
            for k in range(1, D_EDGE):
                ae16 = ae16 + plsc.load_gather(eb, [base4 + k]) * wv[k]
            a = (plsc.load_gather(asrc_t, [s16])
                 + plsc.load_gather(adst_t, [d16])
                 + ae16)
            a = jnp.where(a > 0, a, 0.2 * a)
            wbuf[pl.ds(j * 16, 16)] = jnp.exp(a)
        rb = rows[b]

        def _scale(i4, _):
            for k in range(4):
                i = i4 * 4 + k
                wb = plsc.load_gather(wbuf, [jnp.zeros((16,), jnp.int32) + i])
                for v in range(DAUG // 16):
                    rb[i, pl.ds(v * 16, 16)] = rb[i, pl.ds(v * 16, 16)] * wb
            return 0
        lax.fori_loop(0, C // 4, _scale, 0)
        pltpu.async_copy(rows[b], acc_sp.at[dcs[b]], ssems[b], add=True)

    for d in _gather_descs(0, 0):
        d.start()
    for d in _gather_descs(1, 1):
        d.start()
    _step(0, 0, False, True)

    def _main(t, _):
        g = 3 * t + 1
        _step(g, 1, True, True)
        _step(g + 1, 2, True, True)
        _step(g + 2, 0, True, True)
        return 0
    lax.fori_loop(0, (NCHUNK - 5) // 3, _main, 0)

    _step(NCHUNK - 4, 1, True, True)
    _step(NCHUNK - 3, 2, True, True)
    _step(NCHUNK - 2, 0, True, False)
    _step(NCHUNK - 1, 1, True, False)
    _scatter_desc(1).wait()

    plsc.subcore_barrier()

    pltpu.sync_copy(acc_sp.at[pl.ds(sub * TROWS, TROWS), :],
                    out_hbm.at[core, pl.ds(sub * TROWS, TROWS), :])


def _sc_edge(xp_aug, asrc, adst, src, dst, ea_flat, we_flat, aev):
    mesh = plsc.VectorSubcoreMesh(core_axis_name="c", subcore_axis_name="s")
    f = pl.kernel(
        _sc_edge_body,
        out_type=jax.ShapeDtypeStruct((2, NPAD, DAUG), jnp.float32),
        mesh=mesh,
        compiler_params=pltpu.CompilerParams(needs_layout_passes=False,
                                             use_tc_tiling_on_sc=False),
        scratch_types=[
            pltpu.VMEM((NPAD,), jnp.float32),
            pltpu.VMEM((NPAD,), jnp.float32),
            pltpu.VMEM((EPT,), jnp.int32),
            pltpu.VMEM((EPT,), jnp.int32),
            pltpu.VMEM((D_EDGE * HID,), jnp.float32),
            pltpu.VMEM((HID,), jnp.float32),
            pltpu.VMEM((C,), jnp.float32),
            pltpu.VMEM((C, DAUG), jnp.float32),
            pltpu.VMEM((C, DAUG), jnp.float32),
            pltpu.VMEM((C, DAUG), jnp.float32),
            pltpu.VMEM((C * D_EDGE,), jnp.float32),
            pltpu.VMEM((C * D_EDGE,), jnp.float32),
            pltpu.VMEM((C * D_EDGE,), jnp.float32),
            pltpu.VMEM((C,), jnp.int32),
            pltpu.VMEM((C,), jnp.int32),
            pltpu.VMEM((C,), jnp.int32),
            pltpu.VMEM_SHARED((NPAD, DAUG), jnp.float32),
            pltpu.SemaphoreType.DMA,
            pltpu.SemaphoreType.DMA,
            pltpu.SemaphoreType.DMA,
            pltpu.SemaphoreType.DMA,
            pltpu.SemaphoreType.DMA,
            pltpu.SemaphoreType.DMA,
            pltpu.SemaphoreType.DMA,
        ],
    )
    return f(xp_aug, asrc, adst, src, dst, ea_flat, we_flat, aev)


def kernel(x, edge_index, edge_attr, W1, as1, ad1, We1, ae1, b1,
           W2, as2, ad2, We2, ae2, b2, Wl, bl):
    xpad = jnp.pad(x, ((0, NPAD - N), (0, 0)))
    ea_flat = edge_attr.reshape(E * D_EDGE)
    src = edge_index[0]
    dst = edge_index[1]

    aug1, asr1, adr1 = _feat1(xpad, W1, as1.reshape(HID, 1), ad1.reshape(HID, 1))
    acc1 = _sc_edge(aug1, asr1, adr1, src, dst, ea_flat,
                    We1.reshape(D_EDGE * HID), ae1.reshape(HID))

    aug2, asr2, adr2 = _feat2(acc1, b1.reshape(1, HID), W2,
                              as2.reshape(OUT_CH, 1), ad2.reshape(OUT_CH, 1))
    acc2 = _sc_edge(aug2, asr2, adr2, src, dst, ea_flat,
                    We2.reshape(D_EDGE * OUT_CH), ae2.reshape(OUT_CH))

    out = _final(acc2, b2.reshape(1, OUT_CH), Wl, bl.reshape(1, 1))
    return out[:N]


# speedup vs baseline: 1.2635x; 1.1325x over previous
import functools

import jax
import jax.numpy as jnp
from jax import lax
from jax.experimental import pallas as pl
from jax.experimental.pallas import tpu as pltpu
from jax.experimental.pallas import tpu_sc as plsc

N = 10000
E = 320000
IN_CH = 128
HID = 64
OUT_CH = 64
D_EDGE = 4

NPAD = 10240
DAUG = 80
NTILES = 32
EPT = E // NTILES
C = 80
NCHUNK = EPT // C
TROWS = NPAD // 16

_EPS = 1e-16


def _feat_from_x_body(x_ref, w_ref, asv_ref, adv_ref, aug_ref, as_ref, ad_ref):
    xp = jnp.dot(x_ref[...], w_ref[...], preferred_element_type=jnp.float32)
    aug_ref[:, 0:HID] = xp
    aug_ref[:, HID:HID + 1] = jnp.ones((xp.shape[0], 1), jnp.float32)
    aug_ref[:, HID + 1:DAUG] = jnp.zeros((xp.shape[0], DAUG - HID - 1), jnp.float32)
    as_ref[...] = jnp.dot(xp, asv_ref[...], preferred_element_type=jnp.float32)[:, 0]
    ad_ref[...] = jnp.dot(xp, adv_ref[...], preferred_element_type=jnp.float32)[:, 0]


def _feat1(x, w, asv, adv, blk=2048):
    grid = NPAD // blk
    return pl.pallas_call(
        _feat_from_x_body,
        grid=(grid,),
        in_specs=[
            pl.BlockSpec((blk, IN_CH), lambda i: (i, 0)),
            pl.BlockSpec((IN_CH, HID), lambda i: (0, 0)),
            pl.BlockSpec((HID, 1), lambda i: (0, 0)),
            pl.BlockSpec((HID, 1), lambda i: (0, 0)),
        ],
        out_specs=[
            pl.BlockSpec((blk, DAUG), lambda i: (i, 0)),
            pl.BlockSpec((blk,), lambda i: (i,)),
            pl.BlockSpec((blk,), lambda i: (i,)),
        ],
        out_shape=[
            jax.ShapeDtypeStruct((NPAD, DAUG), jnp.float32),
            jax.ShapeDtypeStruct((NPAD,), jnp.float32),
            jax.ShapeDtypeStruct((NPAD,), jnp.float32),
        ],
    )(x, w, asv, adv)


def _feat_from_acc_body(acc_ref, b_ref, w_ref, asv_ref, adv_ref,
                        aug_ref, as_ref, ad_ref):
    num = acc_ref[0, :, 0:HID] + acc_ref[1, :, 0:HID]
    den = acc_ref[0, :, HID:HID + 1] + acc_ref[1, :, HID:HID + 1]
    h = num / (den + _EPS) + b_ref[...]
    h = jnp.where(h > 0, h, jnp.exp(h) - 1.0)
    xp = jnp.dot(h, w_ref[...], preferred_element_type=jnp.float32)
    aug_ref[:, 0:HID] = xp
    aug_ref[:, HID:HID + 1] = jnp.ones((xp.shape[0], 1), jnp.float32)
    aug_ref[:, HID + 1:DAUG] = jnp.zeros((xp.shape[0], DAUG - HID - 1), jnp.float32)
    as_ref[...] = jnp.dot(xp, asv_ref[...], preferred_element_type=jnp.float32)[:, 0]
    ad_ref[...] = jnp.dot(xp, adv_ref[...], preferred_element_type=jnp.float32)[:, 0]


def _feat2(acc, b, w, asv, adv, blk=2048):
    grid = NPAD // blk
    return pl.pallas_call(
        _feat_from_acc_body,
        grid=(grid,),
        in_specs=[
            pl.BlockSpec((2, blk, DAUG), lambda i: (0, i, 0)),
            pl.BlockSpec((1, HID), lambda i: (0, 0)),
            pl.BlockSpec((HID, HID), lambda i: (0, 0)),
            pl.BlockSpec((HID, 1), lambda i: (0, 0)),
            pl.BlockSpec((HID, 1), lambda i: (0, 0)),
        ],
        out_specs=[
            pl.BlockSpec((blk, DAUG), lambda i: (i, 0)),
            pl.BlockSpec((blk,), lambda i: (i,)),
            pl.BlockSpec((blk,), lambda i: (i,)),
        ],
        out_shape=[
            jax.ShapeDtypeStruct((NPAD, DAUG), jnp.float32),
            jax.ShapeDtypeStruct((NPAD,), jnp.float32),
            jax.ShapeDtypeStruct((NPAD,), jnp.float32),
        ],
    )(acc, b, w, asv, adv)


def _final_body(acc_ref, b_ref, wl_ref, bl_ref, out_ref):
    num = acc_ref[0, :, 0:HID] + acc_ref[1, :, 0:HID]
    den = acc_ref[0, :, HID:HID + 1] + acc_ref[1, :, HID:HID + 1]
    h = num / (den + _EPS) + b_ref[...]
    logit = jnp.dot(h, wl_ref[...], preferred_element_type=jnp.float32) + bl_ref[...]
    out_ref[...] = jax.nn.sigmoid(logit)


def _final(acc, b, wl, bl, blk=2048):
    grid = NPAD // blk
    return pl.pallas_call(
        _final_body,
        grid=(grid,),
        in_specs=[
            pl.BlockSpec((2, blk, DAUG), lambda i: (0, i, 0)),
            pl.BlockSpec((1, OUT_CH), lambda i: (0, 0)),
            pl.BlockSpec((OUT_CH, 1), lambda i: (0, 0)),
            pl.BlockSpec((1, 1), lambda i: (0, 0)),
        ],
        out_specs=pl.BlockSpec((blk, 1), lambda i: (i, 0)),
        out_shape=jax.ShapeDtypeStruct((NPAD, 1), jnp.float32),
    )(acc, b, wl, bl)


def _sc_edge_body(xp_hbm, asrc_hbm, adst_hbm, src_hbm, dst_hbm, ea_hbm,
                  we_hbm, aev_hbm, out_hbm, asrc_t, adst_t, sidx_t, didx_t,
                  we_t, aev_t, wbuf, rows0, rows1, rows2, rows3, rows4,
                  eab0, eab1, eab2, eab3, eab4, dc0, dc1, dc2, dc3, dc4,
                  acc_sp, bsem, gs0, gs1, gs2, gs3, gs4,
                  es0, es1, es2, es3, es4, ss0, ss1, ss2, ss3, ss4):
    core = lax.axis_index("c")
    sub = lax.axis_index("s")
    wid = core * 16 + sub
    ebase = wid * EPT

    rows = [rows0, rows1, rows2, rows3, rows4]
    eabs = [eab0, eab1, eab2, eab3, eab4]
    dcs = [dc0, dc1, dc2, dc3, dc4]
    gsems = [gs0, gs1, gs2, gs3, gs4]
    esems = [es0, es1, es2, es3, es4]
    ssems = [ss0, ss1, ss2, ss3, ss4]
    NB = 5

    zeros16 = jnp.zeros((16,), jnp.float32)
    iota16 = lax.iota(jnp.int32, 16)

    bulk = [
        pltpu.async_copy(src_hbm.at[pl.ds(ebase, EPT)], sidx_t, bsem),
        pltpu.async_copy(dst_hbm.at[pl.ds(ebase, EPT)], didx_t, bsem),
        pltpu.async_copy(asrc_hbm, asrc_t, bsem),
        pltpu.async_copy(adst_hbm, adst_t, bsem),
        pltpu.async_copy(we_hbm, we_t, bsem),
        pltpu.async_copy(aev_hbm, aev_t, bsem),
    ]

    def _zrow(i, _):
        for v in range(DAUG // 16):
            rows0[i, pl.ds(v * 16, 16)] = zeros16
        return 0
    lax.fori_loop(0, C, _zrow, 0)

    def _zinit(k, _):
        pltpu.sync_copy(rows0, acc_sp.at[pl.ds(sub * TROWS + k * C, C), :])
        return 0
    lax.fori_loop(0, TROWS // C, _zinit, 0)

    for d in bulk:
        d.wait()

    wv = []
    for k in range(D_EDGE):
        acc16 = we_t[pl.ds(k * HID, 16)] * aev_t[pl.ds(0, 16)]
        for q in range(1, HID // 16):
            acc16 = acc16 + (we_t[pl.ds(k * HID + q * 16, 16)]
                             * aev_t[pl.ds(q * 16, 16)])
        wv.append(jnp.zeros((16,), jnp.float32) + jnp.sum(acc16))

    plsc.subcore_barrier()

    def _row_desc(g, b):
        return pltpu.make_async_copy(
            xp_hbm.at[sidx_t.at[pl.ds(g * C, C)]], rows[b], gsems[b])

    def _ea_desc(g, b):
        return pltpu.make_async_copy(
            ea_hbm.at[pl.ds((ebase + g * C) * D_EDGE, C * D_EDGE)],
            eabs[b], esems[b])

    def _scatter_desc(b):
        return pltpu.make_async_copy(rows[b], acc_sp.at[dcs[b]], ssems[b])

    def _issue(g, b):
        _row_desc(g, b).start()
        _ea_desc(g, b).start()

    def _step(g, b, wait_prev_scatter, issue_next_gather):
        if wait_prev_scatter:
            _scatter_desc((b + 3) % NB).wait()
        if issue_next_gather:
            _issue(g + 3, (b + 3) % NB)
        _ea_desc(g, b).wait()
        dc = dcs[b]
        eb = eabs[b]
        for j in range(C // 16):
            d16 = didx_t[pl.ds(g * C + j * 16, 16)]
            dc[pl.ds(j * 16, 16)] = d16
            s16 = sidx_t[pl.ds(g * C + j * 16, 16)]
            base4 = (j * 16 + iota16) * D_EDGE
            ae16 = plsc.load_gather(eb, [base4]) * wv[0]
            for k in range(1, D_EDGE):
                ae16 = ae16 + plsc.load_gather(eb, [base4 + k]) * wv[k]
            a = (plsc.load_gather(asrc_t, [s16])
                 + plsc.load_gather(adst_t, [d16])
                 + ae16)
            a = jnp.where(a > 0, a, 0.2 * a)
            wbuf[pl.ds(j * 16, 16)] = jnp.exp(a)
        _row_desc(g, b).wait()
        rb = rows[b]

        def _scale(i4, _):
            for k in range(4):
                i = i4 * 4 + k
                wb = plsc.load_gather(wbuf, [jnp.zeros((16,), jnp.int32) + i])
                for v in range(DAUG // 16):
                    rb[i, pl.ds(v * 16, 16)] = rb[i, pl.ds(v * 16, 16)] * wb
            return 0
        lax.fori_loop(0, C // 4, _scale, 0)
        pltpu.async_copy(rows[b], acc_sp.at[dcs[b]], ssems[b], add=True)

    _issue(0, 0)
    _issue(1, 1)
    _issue(2, 2)
    _step(0, 0, False, True)
    _step(1, 1, False, True)

    def _main(t, _):
        g = NB * t + 2
        _step(g, 2, True, True)
        _step(g + 1, 3, True, True)
        _step(g + 2, 4, True, True)
        _step(g + 3, 0, True, True)
        _step(g + 4, 1, True, True)
        return 0
    lax.fori_loop(0, (NCHUNK - 5) // NB, _main, 0)
    _step(NCHUNK - 3, 2, True, False)
    _step(NCHUNK - 2, 3, True, False)
    _step(NCHUNK - 1, 4, True, False)
    _scatter_desc(3).wait()
    _scatter_desc(4).wait()

    plsc.subcore_barrier()

    pltpu.sync_copy(acc_sp.at[pl.ds(sub * TROWS, TROWS), :],
                    out_hbm.at[core, pl.ds(sub * TROWS, TROWS), :])


def _sc_edge(xp_aug, asrc, adst, src, dst, ea_flat, we_flat, aev):
    mesh = plsc.VectorSubcoreMesh(core_axis_name="c", subcore_axis_name="s")
    f = pl.kernel(
        _sc_edge_body,
        out_type=jax.ShapeDtypeStruct((2, NPAD, DAUG), jnp.float32),
        mesh=mesh,
        compiler_params=pltpu.CompilerParams(needs_layout_passes=False,
                                             use_tc_tiling_on_sc=False),
        scratch_types=[
            pltpu.VMEM((NPAD,), jnp.float32),
            pltpu.VMEM((NPAD,), jnp.float32),
            pltpu.VMEM((EPT,), jnp.int32),
            pltpu.VMEM((EPT,), jnp.int32),
            pltpu.VMEM((D_EDGE * HID,), jnp.float32),
            pltpu.VMEM((HID,), jnp.float32),
            pltpu.VMEM((C,), jnp.float32),
            pltpu.VMEM((C, DAUG), jnp.float32),
            pltpu.VMEM((C, DAUG), jnp.float32),
            pltpu.VMEM((C, DAUG), jnp.float32),
            pltpu.VMEM((C, DAUG), jnp.float32),
            pltpu.VMEM((C, DAUG), jnp.float32),
            pltpu.VMEM((C * D_EDGE,), jnp.float32),
            pltpu.VMEM((C * D_EDGE,), jnp.float32),
            pltpu.VMEM((C * D_EDGE,), jnp.float32),
            pltpu.VMEM((C * D_EDGE,), jnp.float32),
            pltpu.VMEM((C * D_EDGE,), jnp.float32),
            pltpu.VMEM((C,), jnp.int32),
            pltpu.VMEM((C,), jnp.int32),
            pltpu.VMEM((C,), jnp.int32),
            pltpu.VMEM((C,), jnp.int32),
            pltpu.VMEM((C,), jnp.int32),
            pltpu.VMEM_SHARED((NPAD, DAUG), jnp.float32),
            pltpu.SemaphoreType.DMA,
            pltpu.SemaphoreType.DMA,
            pltpu.SemaphoreType.DMA,
            pltpu.SemaphoreType.DMA,
            pltpu.SemaphoreType.DMA,
            pltpu.SemaphoreType.DMA,
            pltpu.SemaphoreType.DMA,
            pltpu.SemaphoreType.DMA,
            pltpu.SemaphoreType.DMA,
            pltpu.SemaphoreType.DMA,
            pltpu.SemaphoreType.DMA,
            pltpu.SemaphoreType.DMA,
            pltpu.SemaphoreType.DMA,
            pltpu.SemaphoreType.DMA,
            pltpu.SemaphoreType.DMA,
            pltpu.SemaphoreType.DMA,
        ],
    )
    return f(xp_aug, asrc, adst, src, dst, ea_flat, we_flat, aev)


def kernel(x, edge_index, edge_attr, W1, as1, ad1, We1, ae1, b1,
           W2, as2, ad2, We2, ae2, b2, Wl, bl):
    xpad = jnp.pad(x, ((0, NPAD - N), (0, 0)))
    ea_flat = edge_attr.reshape(E * D_EDGE)
    src = edge_index[0]
    dst = edge_index[1]

    aug1, asr1, adr1 = _feat1(xpad, W1, as1.reshape(HID, 1), ad1.reshape(HID, 1))
    acc1 = _sc_edge(aug1, asr1, adr1, src, dst, ea_flat,
                    We1.reshape(D_EDGE * HID), ae1.reshape(HID))

    aug2, asr2, adr2 = _feat2(acc1, b1.reshape(1, HID), W2,
                              as2.reshape(OUT_CH, 1), ad2.reshape(OUT_CH, 1))
    acc2 = _sc_edge(aug2, asr2, adr2, src, dst, ea_flat,
                    We2.reshape(D_EDGE * OUT_CH), ae2.reshape(OUT_CH))

    out = _final(acc2, b2.reshape(1, OUT_CH), Wl, bl.reshape(1, 1))
    return out[:N]
